# initial kernel scaffold (unmeasured)
import jax
import jax.numpy as jnp
from jax import lax
from jax.experimental import pallas as pl
from jax.experimental.pallas import tpu as pltpu

N_DEV = 8
S_CHUNK = 512
S_FULL = N_DEV * S_CHUNK
D_MODEL = 1024
N_HEADS = 8
D_HEAD = 128
SCALE = 0.08838834764831843
NSLOT = 3


def kernel(x, Wq, Wo, Wk, Wv):
    x2 = x.reshape(S_CHUNK, D_MODEL)

    def body(x_ref, wq_ref, wo_ref, wk_ref, wv_ref, out_ref,
             q_ref, k_ref, v_ref, ag_comm, rs_comm, attn_ref,
             ag_send, ag_recv, rs_send, rs_recv):
        my = lax.axis_index("i")
        left = lax.rem(my - 1 + N_DEV, N_DEV)
        right = lax.rem(my + 1, N_DEV)

        barrier = pltpu.get_barrier_semaphore()
        for nbr in (left, right):
            pl.semaphore_signal(barrier, inc=1, device_id=(nbr,),
                                device_id_type=pl.DeviceIdType.MESH)
        pl.semaphore_wait(barrier, 2)

        def project(chunk_idx, xc):
            r0 = chunk_idx * S_CHUNK
            q_ref[pl.ds(r0, S_CHUNK), :] = jnp.dot(
                xc, wq_ref[...], preferred_element_type=jnp.float32)
            k_ref[pl.ds(r0, S_CHUNK), :] = jnp.dot(
                xc, wk_ref[...], preferred_element_type=jnp.float32)
            v_ref[pl.ds(r0, S_CHUNK), :] = jnp.dot(
                xc, wv_ref[...], preferred_element_type=jnp.float32)

        ag_comm[0, :, :] = x_ref[...]
        project(my, x_ref[...])

        for h in range(N_DEV - 1):
            s_slot, r_slot = h % NSLOT, (h + 1) % NSLOT
            rdma = pltpu.make_async_remote_copy(
                src_ref=ag_comm.at[s_slot],
                dst_ref=ag_comm.at[r_slot],
                send_sem=ag_send.at[s_slot],
                recv_sem=ag_recv.at[r_slot],
                device_id=(right,),
                device_id_type=pl.DeviceIdType.MESH,
            )
            rdma.start()
            rdma.wait()
            c = lax.rem(my - 1 - h + 2 * N_DEV, N_DEV)
            project(c, ag_comm[r_slot, :, :])

        def chunk_partial(c):
            r0 = c * S_CHUNK

            def head_body(hd, carry):
                col = hd * D_HEAD
                qh = q_ref[pl.ds(r0, S_CHUNK), pl.ds(col, D_HEAD)]
                kh = k_ref[:, pl.ds(col, D_HEAD)]
                vh = v_ref[:, pl.ds(col, D_HEAD)]
                s = lax.dot_general(
                    qh, kh, (((1,), (1,)), ((), ())),
                    preferred_element_type=jnp.float32) * SCALE
                m = jnp.max(s, axis=1, keepdims=True)
                p = jnp.exp(s - m)
                denom = jnp.sum(p, axis=1, keepdims=True)
                o = jnp.dot(p, vh, preferred_element_type=jnp.float32) / denom
                attn_ref[:, pl.ds(col, D_HEAD)] = o
                return carry

            lax.fori_loop(0, N_HEADS, head_body, 0)
            return jnp.dot(attn_ref[...], wo_ref[...],
                           preferred_element_type=jnp.float32)

        for s in range(N_DEV - 1):
            s_slot, r_slot = s % NSLOT, (s + 1) % NSLOT
            c = lax.rem(my - 1 - s + 2 * N_DEV, N_DEV)
            pc = chunk_partial(c)
            if s == 0:
                rs_comm[s_slot, :, :] = pc
            else:
                rs_comm[s_slot, :, :] = pc + rs_comm[s_slot, :, :]
            rdma = pltpu.make_async_remote_copy(
                src_ref=rs_comm.at[s_slot],
                dst_ref=rs_comm.at[r_slot],
                send_sem=rs_send.at[s_slot],
                recv_sem=rs_recv.at[r_slot],
                device_id=(right,),
                device_id_type=pl.DeviceIdType.MESH,
            )
            rdma.start()
            rdma.wait()

        out_ref[...] = chunk_partial(my) + rs_comm[(N_DEV - 1) % NSLOT, :, :]

    out = pl.pallas_call(
        body,
        out_shape=jax.ShapeDtypeStruct((S_CHUNK, D_MODEL), jnp.float32),
        in_specs=[pl.BlockSpec(memory_space=pltpu.VMEM)] * 5,
        out_specs=pl.BlockSpec(memory_space=pltpu.VMEM),
        scratch_shapes=[
            pltpu.VMEM((S_FULL, D_MODEL), jnp.float32),
            pltpu.VMEM((S_FULL, D_MODEL), jnp.float32),
            pltpu.VMEM((S_FULL, D_MODEL), jnp.float32),
            pltpu.VMEM((NSLOT, S_CHUNK, D_MODEL), jnp.float32),
            pltpu.VMEM((NSLOT, S_CHUNK, D_MODEL), jnp.float32),
            pltpu.VMEM((S_CHUNK, D_MODEL), jnp.float32),
            pltpu.SemaphoreType.DMA((NSLOT,)),
            pltpu.SemaphoreType.DMA((NSLOT,)),
            pltpu.SemaphoreType.DMA((NSLOT,)),
            pltpu.SemaphoreType.DMA((NSLOT,)),
        ],
        compiler_params=pltpu.CompilerParams(collective_id=0),
    )(x2, Wq, Wo, Wk, Wv)
    return out.reshape(1, S_CHUNK, D_MODEL)


# baseline (device time: 873082 ns/iter reference)
import jax
import jax.numpy as jnp
from jax import lax
from jax.experimental import pallas as pl
from jax.experimental.pallas import tpu as pltpu

N_DEV = 8
S_CHUNK = 512
S_FULL = N_DEV * S_CHUNK
D_MODEL = 1024
N_HEADS = 8
D_HEAD = 128
Q_BLK = 512
SCALE = 0.08838834764831843
NSLOT = 3


def kernel(x, Wq, Wo, Wk, Wv):
    x2 = x.reshape(S_CHUNK, D_MODEL)

    def body(x_ref, wq_ref, wo_ref, wk_ref, wv_ref, out_ref,
             q_hbm, k_hbm, v_hbm, attn_hbm,
             ag_comm, rs_comm, stage, qh, kh, vh, oh,
             ag_send, ag_recv, rs_send, rs_recv, local_sem):
        my = lax.axis_index("i")
        left = lax.rem(my - 1 + N_DEV, N_DEV)
        right = lax.rem(my + 1, N_DEV)

        barrier = pltpu.get_barrier_semaphore()
        for nbr in (left, right):
            pl.semaphore_signal(barrier, inc=1, device_id=(nbr,),
                                device_id_type=pl.DeviceIdType.MESH)
        pl.semaphore_wait(barrier, 2)

        def to_hbm(dst_hbm, r0, value):
            stage[...] = value
            cp = pltpu.make_async_copy(
                stage, dst_hbm.at[pl.ds(r0, S_CHUNK), :], local_sem)
            cp.start()
            cp.wait()

        def project(chunk_idx, xc):
            r0 = chunk_idx * S_CHUNK
            to_hbm(q_hbm, r0, jnp.dot(xc, wq_ref[...],
                                      preferred_element_type=jnp.float32))
            to_hbm(k_hbm, r0, jnp.dot(xc, wk_ref[...],
                                      preferred_element_type=jnp.float32))
            to_hbm(v_hbm, r0, jnp.dot(xc, wv_ref[...],
                                      preferred_element_type=jnp.float32))

        ag_comm[0, :, :] = x_ref[...]
        project(my, x_ref[...])
        for h in range(N_DEV - 1):
            s_slot, r_slot = h % NSLOT, (h + 1) % NSLOT
            rdma = pltpu.make_async_remote_copy(
                src_ref=ag_comm.at[s_slot],
                dst_ref=ag_comm.at[r_slot],
                send_sem=ag_send.at[s_slot],
                recv_sem=ag_recv.at[r_slot],
                device_id=(right,),
                device_id_type=pl.DeviceIdType.MESH,
            )
            rdma.start()
            rdma.wait()
            c = lax.rem(my - 1 - h + 2 * N_DEV, N_DEV)
            project(c, ag_comm[r_slot, :, :])

        def head_body(hd, carry):
            col = hd * D_HEAD
            cps = [
                pltpu.make_async_copy(
                    src.at[:, pl.ds(col, D_HEAD)], dst, local_sem)
                for src, dst in ((q_hbm, qh), (k_hbm, kh), (v_hbm, vh))
            ]
            for cp in cps:
                cp.start()
                cp.wait()

            def q_body(qb, carry2):
                r0 = qb * Q_BLK
                qblk = qh[pl.ds(r0, Q_BLK), :]
                s = lax.dot_general(
                    qblk, kh[...], (((1,), (1,)), ((), ())),
                    preferred_element_type=jnp.float32) * SCALE
                m = jnp.max(s, axis=1, keepdims=True)
                p = jnp.exp(s - m)
                denom = jnp.sum(p, axis=1, keepdims=True)
                oh[pl.ds(r0, Q_BLK), :] = jnp.dot(
                    p, vh[...], preferred_element_type=jnp.float32) / denom
                return carry2

            lax.fori_loop(0, S_FULL // Q_BLK, q_body, 0)
            cp = pltpu.make_async_copy(
                oh, attn_hbm.at[:, pl.ds(col, D_HEAD)], local_sem)
            cp.start()
            cp.wait()
            return carry

        lax.fori_loop(0, N_HEADS, head_body, 0)

        def chunk_partial(c):
            cp = pltpu.make_async_copy(
                attn_hbm.at[pl.ds(c * S_CHUNK, S_CHUNK), :], stage, local_sem)
            cp.start()
            cp.wait()
            return jnp.dot(stage[...], wo_ref[...],
                           preferred_element_type=jnp.float32)

        for s in range(N_DEV - 1):
            s_slot, r_slot = s % NSLOT, (s + 1) % NSLOT
            c = lax.rem(my - 1 - s + 2 * N_DEV, N_DEV)
            pc = chunk_partial(c)
            if s == 0:
                rs_comm[s_slot, :, :] = pc
            else:
                rs_comm[s_slot, :, :] = pc + rs_comm[s_slot, :, :]
            rdma = pltpu.make_async_remote_copy(
                src_ref=rs_comm.at[s_slot],
                dst_ref=rs_comm.at[r_slot],
                send_sem=rs_send.at[s_slot],
                recv_sem=rs_recv.at[r_slot],
                device_id=(right,),
                device_id_type=pl.DeviceIdType.MESH,
            )
            rdma.start()
            rdma.wait()

        out_ref[...] = chunk_partial(my) + rs_comm[(N_DEV - 1) % NSLOT, :, :]

    hbm_scratch = jax.ShapeDtypeStruct((S_FULL, D_MODEL), jnp.float32)
    out = pl.pallas_call(
        body,
        out_shape=(
            jax.ShapeDtypeStruct((S_CHUNK, D_MODEL), jnp.float32),
            hbm_scratch,
            hbm_scratch,
            hbm_scratch,
            hbm_scratch,
        ),
        in_specs=[pl.BlockSpec(memory_space=pltpu.VMEM)] * 5,
        out_specs=(
            pl.BlockSpec(memory_space=pltpu.VMEM),
            pl.BlockSpec(memory_space=pltpu.HBM),
            pl.BlockSpec(memory_space=pltpu.HBM),
            pl.BlockSpec(memory_space=pltpu.HBM),
            pl.BlockSpec(memory_space=pltpu.HBM),
        ),
        scratch_shapes=[
            pltpu.VMEM((NSLOT, S_CHUNK, D_MODEL), jnp.float32),
            pltpu.VMEM((NSLOT, S_CHUNK, D_MODEL), jnp.float32),
            pltpu.VMEM((S_CHUNK, D_MODEL), jnp.float32),
            pltpu.VMEM((S_FULL, D_HEAD), jnp.float32),
            pltpu.VMEM((S_FULL, D_HEAD), jnp.float32),
            pltpu.VMEM((S_FULL, D_HEAD), jnp.float32),
            pltpu.VMEM((S_FULL, D_HEAD), jnp.float32),
            pltpu.SemaphoreType.DMA((NSLOT,)),
            pltpu.SemaphoreType.DMA((NSLOT,)),
            pltpu.SemaphoreType.DMA((NSLOT,)),
            pltpu.SemaphoreType.DMA((NSLOT,)),
            pltpu.SemaphoreType.DMA,
        ],
        compiler_params=pltpu.CompilerParams(collective_id=0),
    )(x2, Wq, Wo, Wk, Wv)
    return out[0].reshape(1, S_CHUNK, D_MODEL)


# device time: 789198 ns/iter; 1.1063x vs baseline; 1.1063x over previous
import jax
import jax.numpy as jnp
from jax import lax
from jax.experimental import pallas as pl
from jax.experimental.pallas import tpu as pltpu

N_DEV = 8
S_CHUNK = 512
S_FULL = N_DEV * S_CHUNK
D_MODEL = 1024
N_HEADS = 8
D_HEAD = 128
Q_BLK = 512
SCALE = 0.08838834764831843
NSLOT = 3


def kernel(x, Wq, Wo, Wk, Wv):
    x2 = x.reshape(S_CHUNK, D_MODEL)

    def body(x_ref, wq_ref, wo_ref, wk_ref, wv_ref, out_ref,
             q_hbm, k_hbm, v_hbm, attn_hbm,
             ag_comm, rs_comm, stage, qh, kh, vh, oh,
             ag_send, ag_recv, rs_send, rs_recv, local_sem):
        my = lax.axis_index("i")
        left = lax.rem(my - 1 + N_DEV, N_DEV)
        right = lax.rem(my + 1, N_DEV)

        barrier = pltpu.get_barrier_semaphore()
        for nbr in (left, right):
            pl.semaphore_signal(barrier, inc=1, device_id=(nbr,),
                                device_id_type=pl.DeviceIdType.MESH)
        pl.semaphore_wait(barrier, 2)

        def to_hbm(dst_hbm, r0, value):
            stage[...] = value
            cp = pltpu.make_async_copy(
                stage, dst_hbm.at[pl.ds(r0, S_CHUNK), :], local_sem.at[0])
            cp.start()
            cp.wait()

        def project(chunk_idx, xc):
            r0 = chunk_idx * S_CHUNK
            to_hbm(q_hbm, r0, jnp.dot(xc, wq_ref[...],
                                      preferred_element_type=jnp.float32))
            to_hbm(k_hbm, r0, jnp.dot(xc, wk_ref[...],
                                      preferred_element_type=jnp.float32))
            to_hbm(v_hbm, r0, jnp.dot(xc, wv_ref[...],
                                      preferred_element_type=jnp.float32))

        ag_comm[0, :, :] = x_ref[...]
        for h in range(N_DEV - 1):
            s_slot, r_slot = h % NSLOT, (h + 1) % NSLOT
            rdma = pltpu.make_async_remote_copy(
                src_ref=ag_comm.at[s_slot],
                dst_ref=ag_comm.at[r_slot],
                send_sem=ag_send.at[s_slot],
                recv_sem=ag_recv.at[r_slot],
                device_id=(right,),
                device_id_type=pl.DeviceIdType.MESH,
            )
            rdma.start()
            if h == 0:
                project(my, x_ref[...])
            else:
                c = lax.rem(my - h + 2 * N_DEV, N_DEV)
                project(c, ag_comm[s_slot, :, :])
            rdma.wait()
        project(lax.rem(my + 1, N_DEV),
                ag_comm[(N_DEV - 1) % NSLOT, :, :])

        def head_body(hd, carry):
            col = hd * D_HEAD
            cps = [
                pltpu.make_async_copy(
                    src.at[:, pl.ds(col, D_HEAD)], dst, local_sem.at[i])
                for i, (src, dst) in enumerate(
                    ((q_hbm, qh), (k_hbm, kh), (v_hbm, vh)))
            ]
            for cp in cps:
                cp.start()
            for cp in cps:
                cp.wait()

            def q_body(qb, carry2):
                r0 = qb * Q_BLK
                qblk = qh[pl.ds(r0, Q_BLK), :]
                s = lax.dot_general(
                    qblk, kh[...], (((1,), (1,)), ((), ())),
                    preferred_element_type=jnp.float32) * SCALE
                m = jnp.max(s, axis=1, keepdims=True)
                p = jnp.exp(s - m)
                denom = jnp.sum(p, axis=1, keepdims=True)
                oh[pl.ds(r0, Q_BLK), :] = jnp.dot(
                    p, vh[...], preferred_element_type=jnp.float32) / denom
                return carry2

            lax.fori_loop(0, S_FULL // Q_BLK, q_body, 0)
            cp = pltpu.make_async_copy(
                oh, attn_hbm.at[:, pl.ds(col, D_HEAD)], local_sem.at[3])
            cp.start()
            cp.wait()
            return carry

        lax.fori_loop(0, N_HEADS, head_body, 0)

        def chunk_partial(c):
            cp = pltpu.make_async_copy(
                attn_hbm.at[pl.ds(c * S_CHUNK, S_CHUNK), :], stage,
                local_sem.at[0])
            cp.start()
            cp.wait()
            return jnp.dot(stage[...], wo_ref[...],
                           preferred_element_type=jnp.float32)

        prev = None
        for s in range(N_DEV - 1):
            s_slot, r_slot = s % NSLOT, (s + 1) % NSLOT
            c = lax.rem(my - 1 - s + 2 * N_DEV, N_DEV)
            pc = chunk_partial(c)
            if prev is not None:
                prev.wait()
            if s == 0:
                rs_comm[s_slot, :, :] = pc
            else:
                rs_comm[s_slot, :, :] = pc + rs_comm[s_slot, :, :]
            rdma = pltpu.make_async_remote_copy(
                src_ref=rs_comm.at[s_slot],
                dst_ref=rs_comm.at[r_slot],
                send_sem=rs_send.at[s_slot],
                recv_sem=rs_recv.at[r_slot],
                device_id=(right,),
                device_id_type=pl.DeviceIdType.MESH,
            )
            rdma.start()
            prev = rdma

        pc = chunk_partial(my)
        prev.wait()
        out_ref[...] = pc + rs_comm[(N_DEV - 1) % NSLOT, :, :]

    hbm_scratch = jax.ShapeDtypeStruct((S_FULL, D_MODEL), jnp.float32)
    out = pl.pallas_call(
        body,
        out_shape=(
            jax.ShapeDtypeStruct((S_CHUNK, D_MODEL), jnp.float32),
            hbm_scratch,
            hbm_scratch,
            hbm_scratch,
            hbm_scratch,
        ),
        in_specs=[pl.BlockSpec(memory_space=pltpu.VMEM)] * 5,
        out_specs=(
            pl.BlockSpec(memory_space=pltpu.VMEM),
            pl.BlockSpec(memory_space=pltpu.HBM),
            pl.BlockSpec(memory_space=pltpu.HBM),
            pl.BlockSpec(memory_space=pltpu.HBM),
            pl.BlockSpec(memory_space=pltpu.HBM),
        ),
        scratch_shapes=[
            pltpu.VMEM((NSLOT, S_CHUNK, D_MODEL), jnp.float32),
            pltpu.VMEM((NSLOT, S_CHUNK, D_MODEL), jnp.float32),
            pltpu.VMEM((S_CHUNK, D_MODEL), jnp.float32),
            pltpu.VMEM((S_FULL, D_HEAD), jnp.float32),
            pltpu.VMEM((S_FULL, D_HEAD), jnp.float32),
            pltpu.VMEM((S_FULL, D_HEAD), jnp.float32),
            pltpu.VMEM((S_FULL, D_HEAD), jnp.float32),
            pltpu.SemaphoreType.DMA((NSLOT,)),
            pltpu.SemaphoreType.DMA((NSLOT,)),
            pltpu.SemaphoreType.DMA((NSLOT,)),
            pltpu.SemaphoreType.DMA((NSLOT,)),
            pltpu.SemaphoreType.DMA((4,)),
        ],
        compiler_params=pltpu.CompilerParams(
            collective_id=0,
            vmem_limit_bytes=56 * 1024 * 1024,
        ),
    )(x2, Wq, Wo, Wk, Wv)
    return out[0].reshape(1, S_CHUNK, D_MODEL)
